# baseline (speedup 1.0000x reference)
import functools

import jax
import jax.numpy as jnp
from jax.experimental import pallas as pl
from jax.experimental.pallas import tpu as pltpu

_ALPHA = 10.0
_BETA = 2.0


def _fused_loss_kernel(x_full_ref, k_ref, t_full_ref,
                       t_row_ref, acc_ref, chi_ref, clo_ref, clo2_ref, cnt_ref,
                       *, n_classes, tile):
    i = pl.program_id(0)
    x_tile = x_full_ref[pl.ds(i * tile, tile), :]
    t_col = t_full_ref[pl.ds(i * tile, tile), :]
    cp = chi_ref.shape[0]

    @pl.when(i == 0)
    def _build_class_sums():
        x_full = x_full_ref[...]
        t_full = t_full_ref[...]
        n = x_full.shape[0]
        col_n = jax.lax.broadcasted_iota(jnp.int32, (n, cp), 1)
        oh = jnp.where(col_n == t_full, 1.0, 0.0).astype(jnp.bfloat16)
        cnt_ref[...] = jnp.sum(oh.astype(jnp.float32), axis=0, keepdims=True)
        c_f32 = jax.lax.dot_general(oh, x_full, (((0,), (0,)), ((), ())),
                                    preferred_element_type=jnp.float32)
        c_hi = c_f32.astype(jnp.bfloat16)
        r1 = c_f32 - c_hi.astype(jnp.float32)
        c_lo = r1.astype(jnp.bfloat16)
        c_lo2 = (r1 - c_lo.astype(jnp.float32)).astype(jnp.bfloat16)
        chi_ref[...] = c_hi
        clo_ref[...] = c_lo
        clo2_ref[...] = c_lo2

    def dotc(cpart):
        return jax.lax.dot_general(x_tile, cpart, (((1,), (1,)), ((), ())),
                                   preferred_element_type=jnp.float32)

    d_mat = dotc(chi_ref[...]) + dotc(clo_ref[...]) + dotc(clo2_ref[...])

    sim = jax.lax.dot_general(x_tile, x_full_ref[...], (((1,), (1,)), ((), ())),
                              preferred_element_type=jnp.float32)
    same = t_col == t_row_ref[...]
    pos_mask = jnp.logical_and(same, sim < 1.0)
    pos_cnt = jnp.sum(pos_mask.astype(jnp.float32), axis=1, keepdims=True)
    pos_sum = jnp.sum(jnp.where(pos_mask, sim, 0.0), axis=1, keepdims=True)

    col = jax.lax.broadcasted_iota(jnp.int32, (tile, cp), 1)
    is_tgt = col == t_col

    def gather_tgt(mat):
        return jnp.sum(jnp.where(is_tgt, mat, 0.0), axis=1, keepdims=True)

    same_sum = gather_tgt(d_mat)
    tot_sum = jnp.sum(d_mat, axis=1, keepdims=True)
    same_cnt = gather_tgt(jnp.broadcast_to(cnt_ref[...], is_tgt.shape))
    neg_cnt = jnp.float32(x_full_ref.shape[0]) - same_cnt
    neg_sum = tot_sum - same_sum
    pos_part = jnp.sum(pos_sum / pos_cnt)
    neg_part = jnp.sum(neg_sum / neg_cnt)

    kmat = k_ref[...]
    norm2 = jnp.sum(kmat * kmat, axis=0, keepdims=True)
    kn = (kmat * jax.lax.rsqrt(jnp.maximum(norm2, 1e-30))).astype(jnp.bfloat16)
    cos = jax.lax.dot_general(x_tile, kn, (((1,), (0,)), ((), ())),
                              preferred_element_type=jnp.float32)
    cos = jnp.clip(cos, -1.0, 1.0)
    logits = jnp.where(is_tgt, cos - _BETA, cos) * _ALPHA
    logits = jnp.where(col < n_classes, logits, -1e30)
    m = jnp.max(logits, axis=1, keepdims=True)
    lse = m + jnp.log(jnp.sum(jnp.exp(logits - m), axis=1, keepdims=True))
    tgt_logit = gather_tgt(logits)
    loss_part = jnp.sum(lse - tgt_logit)
    pred = jnp.min(jnp.where(logits == m, col, jnp.int32(2**30)),
                   axis=1, keepdims=True)
    prec_part = jnp.sum((pred == t_col).astype(jnp.float32))

    row = jax.lax.broadcasted_iota(jnp.int32, acc_ref.shape, 0)
    partials = jnp.where(
        row == 0, loss_part,
        jnp.where(row == 1, prec_part,
                  jnp.where(row == 2, pos_part,
                            jnp.where(row == 3, neg_part, 0.0))))

    @pl.when(i == 0)
    def _init():
        acc_ref[...] = partials

    @pl.when(i != 0)
    def _accum():
        acc_ref[...] += partials


def kernel(inputs, targets, kmat):
    n, d = inputs.shape
    c = kmat.shape[1]
    cp = (c + 127) // 128 * 128
    tile = 2048
    grid = n // tile

    x_bf = inputs.astype(jnp.bfloat16)
    k_pad = jnp.pad(kmat, ((0, 0), (0, cp - c)))
    t_col = targets.reshape(n, 1)

    acc = pl.pallas_call(
        functools.partial(_fused_loss_kernel, n_classes=c, tile=tile),
        grid=(grid,),
        in_specs=[
            pl.BlockSpec((n, d), lambda i: (0, 0)),
            pl.BlockSpec((d, cp), lambda i: (0, 0)),
            pl.BlockSpec((n, 1), lambda i: (0, 0)),
            pl.BlockSpec((1, n), lambda i: (0, 0)),
        ],
        out_specs=pl.BlockSpec((8, 128), lambda i: (0, 0)),
        out_shape=jax.ShapeDtypeStruct((8, 128), jnp.float32),
        scratch_shapes=[
            pltpu.VMEM((cp, d), jnp.bfloat16),
            pltpu.VMEM((cp, d), jnp.bfloat16),
            pltpu.VMEM((cp, d), jnp.bfloat16),
            pltpu.VMEM((1, cp), jnp.float32),
        ],
        compiler_params=pltpu.CompilerParams(
            dimension_semantics=("arbitrary",)),
    )(x_bf, k_pad, t_col, targets.reshape(1, n))

    nf = jnp.float32(n)
    return (acc[0, 0] / nf, acc[1, 0] / nf, acc[2, 0] / nf, acc[3, 0] / nf)


# speedup vs baseline: 1.0441x; 1.0441x over previous
import functools

import jax
import jax.numpy as jnp
from jax.experimental import pallas as pl
from jax.experimental.pallas import tpu as pltpu

_ALPHA = 10.0
_BETA = 2.0


def _fused_loss_kernel(x_full_ref, k_ref, t_full_ref,
                       t_row_ref, acc_ref, chi_ref, clo_ref, clo2_ref, cnt_ref,
                       *, n_classes, tile):
    i = pl.program_id(0)
    x_tile = x_full_ref[pl.ds(i * tile, tile), :]
    t_col = t_full_ref[pl.ds(i * tile, tile), :]
    cp = chi_ref.shape[0]

    @pl.when(i == 0)
    def _build_class_sums():
        x_full = x_full_ref[...]
        t_full = t_full_ref[...]
        n = x_full.shape[0]
        col_n = jax.lax.broadcasted_iota(jnp.int32, (n, cp), 1)
        oh = jnp.where(col_n == t_full, 1.0, 0.0).astype(jnp.bfloat16)
        cnt_ref[...] = jnp.sum(oh.astype(jnp.float32), axis=0, keepdims=True)
        c_f32 = jax.lax.dot_general(oh, x_full, (((0,), (0,)), ((), ())),
                                    preferred_element_type=jnp.float32)
        c_hi = c_f32.astype(jnp.bfloat16)
        r1 = c_f32 - c_hi.astype(jnp.float32)
        c_lo = r1.astype(jnp.bfloat16)
        c_lo2 = (r1 - c_lo.astype(jnp.float32)).astype(jnp.bfloat16)
        chi_ref[...] = c_hi
        clo_ref[...] = c_lo
        clo2_ref[...] = c_lo2

    def dotc(cpart):
        return jax.lax.dot_general(x_tile, cpart, (((1,), (1,)), ((), ())),
                                   preferred_element_type=jnp.float32)

    d_mat = dotc(chi_ref[...]) + dotc(clo_ref[...]) + dotc(clo2_ref[...])

    sim = jax.lax.dot_general(x_tile, x_full_ref[...], (((1,), (1,)), ((), ())),
                              preferred_element_type=jnp.float32)
    same = t_col == t_row_ref[...]
    pos_mask = jnp.logical_and(same, sim < 1.0)
    pos_cnt = jnp.sum(pos_mask.astype(jnp.float32), axis=1, keepdims=True)
    pos_sum = jnp.sum(jnp.where(pos_mask, sim, 0.0), axis=1, keepdims=True)

    col = jax.lax.broadcasted_iota(jnp.int32, (tile, cp), 1)
    is_tgt = col == t_col

    def gather_tgt(mat):
        return jnp.sum(jnp.where(is_tgt, mat, 0.0), axis=1, keepdims=True)

    same_sum = gather_tgt(d_mat)
    tot_sum = jnp.sum(d_mat, axis=1, keepdims=True)
    same_cnt = gather_tgt(jnp.broadcast_to(cnt_ref[...], is_tgt.shape))
    neg_cnt = jnp.float32(x_full_ref.shape[0]) - same_cnt
    neg_sum = tot_sum - same_sum
    pos_part = jnp.sum(pos_sum / pos_cnt)
    neg_part = jnp.sum(neg_sum / neg_cnt)

    kmat = k_ref[...]
    norm2 = jnp.sum(kmat * kmat, axis=0, keepdims=True)
    kn = (kmat * jax.lax.rsqrt(jnp.maximum(norm2, 1e-30))).astype(jnp.bfloat16)
    cos = jax.lax.dot_general(x_tile, kn, (((1,), (0,)), ((), ())),
                              preferred_element_type=jnp.float32)
    cos = jnp.clip(cos, -1.0, 1.0)
    logits = jnp.where(is_tgt, cos - _BETA, cos) * _ALPHA
    logits = jnp.where(col < n_classes, logits, -1e30)
    m = jnp.max(logits, axis=1, keepdims=True)
    lse = m + jnp.log(jnp.sum(jnp.exp(logits - m), axis=1, keepdims=True))
    tgt_logit = gather_tgt(logits)
    loss_part = jnp.sum(lse - tgt_logit)
    pred = jnp.min(jnp.where(logits == m, col, jnp.int32(2**30)),
                   axis=1, keepdims=True)
    prec_part = jnp.sum((pred == t_col).astype(jnp.float32))

    row = jax.lax.broadcasted_iota(jnp.int32, acc_ref.shape, 0)
    partials = jnp.where(
        row == 0, loss_part,
        jnp.where(row == 1, prec_part,
                  jnp.where(row == 2, pos_part,
                            jnp.where(row == 3, neg_part, 0.0))))

    @pl.when(i == 0)
    def _init():
        acc_ref[...] = partials

    @pl.when(i != 0)
    def _accum():
        acc_ref[...] += partials


def kernel(inputs, targets, kmat):
    n, d = inputs.shape
    c = kmat.shape[1]
    cp = (c + 127) // 128 * 128
    tile = 1024
    grid = n // tile

    x_bf = inputs.astype(jnp.bfloat16)
    k_pad = jnp.pad(kmat, ((0, 0), (0, cp - c)))
    t_col = targets.reshape(n, 1)

    acc = pl.pallas_call(
        functools.partial(_fused_loss_kernel, n_classes=c, tile=tile),
        grid=(grid,),
        in_specs=[
            pl.BlockSpec((n, d), lambda i: (0, 0)),
            pl.BlockSpec((d, cp), lambda i: (0, 0)),
            pl.BlockSpec((n, 1), lambda i: (0, 0)),
            pl.BlockSpec((1, n), lambda i: (0, 0)),
        ],
        out_specs=pl.BlockSpec((8, 128), lambda i: (0, 0)),
        out_shape=jax.ShapeDtypeStruct((8, 128), jnp.float32),
        scratch_shapes=[
            pltpu.VMEM((cp, d), jnp.bfloat16),
            pltpu.VMEM((cp, d), jnp.bfloat16),
            pltpu.VMEM((cp, d), jnp.bfloat16),
            pltpu.VMEM((1, cp), jnp.float32),
        ],
        compiler_params=pltpu.CompilerParams(
            dimension_semantics=("arbitrary",)),
    )(x_bf, k_pad, t_col, targets.reshape(1, n))

    nf = jnp.float32(n)
    return (acc[0, 0] / nf, acc[1, 0] / nf, acc[2, 0] / nf, acc[3, 0] / nf)
